# baseline (speedup 1.0000x reference)
import functools
import jax
import jax.numpy as jnp
from jax import lax
from jax.experimental import pallas as pl
from jax.experimental.pallas import tpu as pltpu
from jax.experimental.pallas import tpu_sc as plsc

LANES = 128
NC, NS = 2, 16
NW = NC * NS


def _sc_ohem(x, trm, tnm):
    b, _, pixb = x.shape
    total = b * pixb
    per_w = total // NW
    wpb = pixb // per_w
    ch = min(4096, per_w)
    nch = per_w // ch
    mesh = plsc.VectorSubcoreMesh(core_axis_name="c", subcore_axis_name="s")

    @functools.partial(
        pl.kernel, mesh=mesh,
        out_type=[
            jax.ShapeDtypeStruct((NW, 4, 16), jnp.float32),
            jax.ShapeDtypeStruct((total,), jnp.float32),
        ],
        scratch_types=[
            pltpu.VMEM((ch,), jnp.float32),
            pltpu.VMEM((ch,), jnp.float32),
            pltpu.VMEM((ch,), jnp.int32),
            pltpu.VMEM((ch,), jnp.int32),
            pltpu.VMEM((ch,), jnp.float32),
            pltpu.VMEM((4, 16), jnp.float32),
        ],
    )
    def k(x_hbm, tr_hbm, tn_hbm, out_hbm, ce_hbm,
          l0_v, l1_v, tr_v, tn_v, ce_v, st_v):
        wid = lax.axis_index("s") * NC + lax.axis_index("c")
        bb = wid // wpb
        off = (wid % wpb) * per_w

        def chunk_body(ci, carry):
            start = off + ci * ch
            pltpu.sync_copy(x_hbm.at[bb, 0, pl.ds(start, ch)], l0_v)
            pltpu.sync_copy(x_hbm.at[bb, 1, pl.ds(start, ch)], l1_v)
            pltpu.sync_copy(tr_hbm.at[bb, pl.ds(start, ch)], tr_v)
            pltpu.sync_copy(tn_hbm.at[bb, pl.ds(start, ch)], tn_v)

            def vec_body(i, c2):
                np2, nn2, lp2, ns2 = c2
                sl = pl.ds(i * 16, 16)
                a = l0_v[sl]
                bq = l1_v[sl]
                t = tr_v[sl]
                n = tn_v[sl]
                posv = (t * n).astype(jnp.float32)
                negv = ((1 - t) * n).astype(jnp.float32)
                m = jnp.maximum(a, bq)
                u = jnp.exp(-jnp.abs(a - bq))
                z = u / (2.0 + u)
                z2 = z * z
                lnw = 2.0 * z * (1.0 + z2 * (
                    (1.0 / 3.0) + z2 * (0.2 + z2 * (1.0 / 7.0))))
                ce = m - jnp.where(t == 1, bq, a) + lnw
                ce_v[sl] = jnp.where(negv > 0.0, ce, -1.0)
                return (np2 + posv, nn2 + negv,
                        lp2 + posv * ce, ns2 + negv * ce)

            res = lax.fori_loop(0, ch // 16, vec_body, carry)
            pltpu.sync_copy(ce_v, ce_hbm.at[pl.ds(bb * pixb + start, ch)])
            return res

        z16 = jnp.zeros((16,), jnp.float32)
        np_, nn_, lp_, ns_ = lax.fori_loop(
            0, nch, chunk_body, (z16, z16, z16, z16))
        st_v[0] = np_
        st_v[1] = nn_
        st_v[2] = lp_
        st_v[3] = ns_
        pltpu.sync_copy(st_v, out_hbm.at[wid])

    return k(x, trm, tnm)


def _sl1(r):
    d = jnp.abs(r - 1.0)
    return jnp.where(d < 1.0, 0.5 * d * d, d - 0.5)


def _sl1_custom(x, t):
    d = jnp.abs(x - t)
    return jnp.where(d < 1.0 / 9.0, 4.5 * d * d, d - 1.0 / 18.0)


def _ce2(a, b, label):
    m = jnp.maximum(a, b)
    sp = jnp.log(1.0 + jnp.exp(-jnp.abs(a - b)))
    chosen = jnp.where(label == 1, b, a)
    return m - chosen + sp


def _make_dense_body(nblk, rb):
    def body(x23_ref, x45_ref, x67_ref, tr_ref, tn_ref, tcl_ref, rad_ref,
             sin_ref, cos_ref, out_ref, acc_ref):
        b = pl.program_id(0)
        j = pl.program_id(1)
        nb = pl.num_programs(0)
        step = b * nblk + j

        @pl.when(step == 0)
        def _init():
            acc_ref[...] = jnp.zeros_like(acc_ref)

        trm = tr_ref[0]
        tnm = tn_ref[0]
        tclm = tcl_ref[0]
        posf = (trm * tnm).astype(jnp.float32)

        ce_tcl = _ce2(x23_ref[0, 0], x23_ref[0, 1], tclm)

        sn = x45_ref[0, 0]
        cs = x45_ref[0, 1]
        scale = jax.lax.rsqrt(sn * sn + cs * cs + 0.0001)
        snp = sn * scale
        csp = cs * scale

        tp = x67_ref[0, 0]
        bt = x67_ref[0, 1]
        topm = rad_ref[0, 0]
        botm = rad_ref[0, 1]
        rad_l = _sl1(tp / (topm + 0.01)) + _sl1(bt / (botm + 0.01))

        tcl_sel = tclm == 1
        s0 = posf * ce_tcl
        s1 = tclm.astype(jnp.float32)
        s2 = (tnm * tclm).astype(jnp.float32)
        s3 = jnp.where(tcl_sel, rad_l, 0.0)
        s4 = jnp.where(tcl_sel, _sl1_custom(snp, sin_ref[0]), 0.0)
        s5 = jnp.where(tcl_sel, _sl1_custom(csp, cos_ref[0]), 0.0)

        for q, s in enumerate((s0, s1, s2, s3, s4, s5)):
            acc_ref[q * 8:(q + 1) * 8, :] += jnp.sum(
                s.reshape(rb // 8, 8, LANES), axis=0)

        @pl.when(step == nb * nblk - 1)
        def _fin():
            vals = [jnp.sum(acc_ref[q * 8:(q + 1) * 8, :]) for q in range(6)]
            lane = jax.lax.broadcasted_iota(jnp.int32, (1, LANES), 1)
            outv = jnp.zeros((1, LANES), jnp.float32)
            for i, v in enumerate(vals):
                outv = outv + jnp.where(lane == i, v, 0.0)
            out_ref[...] = outv

    return body


def _tc_dense(x, trm, tnm, tcl, rad, snm, csm):
    b = x.shape[0]
    rows_b = x.shape[2]
    rb = min(512, rows_b)
    nblk = rows_b // rb
    body = _make_dense_body(nblk, rb)
    return pl.pallas_call(
        body,
        grid=(b, nblk),
        in_specs=[
            pl.BlockSpec((1, 2, rb, LANES), lambda i, j: (i, 1, j, 0)),
            pl.BlockSpec((1, 2, rb, LANES), lambda i, j: (i, 2, j, 0)),
            pl.BlockSpec((1, 2, rb, LANES), lambda i, j: (i, 3, j, 0)),
            pl.BlockSpec((1, rb, LANES), lambda i, j: (i, j, 0)),
            pl.BlockSpec((1, rb, LANES), lambda i, j: (i, j, 0)),
            pl.BlockSpec((1, rb, LANES), lambda i, j: (i, j, 0)),
            pl.BlockSpec((1, 2, rb, LANES), lambda i, j: (i, 0, j, 0)),
            pl.BlockSpec((1, rb, LANES), lambda i, j: (i, j, 0)),
            pl.BlockSpec((1, rb, LANES), lambda i, j: (i, j, 0)),
        ],
        out_specs=pl.BlockSpec((1, LANES), lambda i, j: (0, 0)),
        out_shape=jax.ShapeDtypeStruct((1, LANES), jnp.float32),
        scratch_shapes=[pltpu.VMEM((48, LANES), jnp.float32)],
    )(x, x, x, trm, tnm, tcl, rad, snm, csm)


def _make_combine_body(rows_total, g_rows):
    def body(scp_ref, tcp_ref, gp_ref, gl_ref, ce_hbm_ref, out_ref,
             ce_scr, topk_ref, sem):
        v = scp_ref[...]
        lane2 = jax.lax.broadcasted_iota(jnp.int32, (16, LANES), 1)
        q = (lane2 % 64) // 16
        n_pos = jnp.sum(jnp.where(q == 0, v, 0.0))
        n_neg_avail = jnp.sum(jnp.where(q == 1, v, 0.0))
        loss_pos = jnp.sum(jnp.where(q == 2, v, 0.0))
        neg_sum = jnp.sum(jnp.where(q == 3, v, 0.0))

        tv = tcp_ref[...]
        lane1 = jax.lax.broadcasted_iota(jnp.int32, (1, LANES), 1)
        tcl_sum = jnp.sum(jnp.where(lane1 == 0, tv, 0.0))
        n_tcl_sel = jnp.sum(jnp.where(lane1 == 1, tv, 0.0))
        n_tcl_train = jnp.sum(jnp.where(lane1 == 2, tv, 0.0))
        radii_sum = jnp.sum(jnp.where(lane1 == 3, tv, 0.0))
        sin_sum = jnp.sum(jnp.where(lane1 == 4, tv, 0.0))
        cos_sum = jnp.sum(jnp.where(lane1 == 5, tv, 0.0))

        n_neg = jnp.where(
            n_pos > 0.0,
            jnp.minimum(n_neg_avail, jnp.floor(3.0 * n_pos)),
            jnp.float32(100.0))

        topk_ref[0] = neg_sum

        @pl.when(n_neg < n_neg_avail)
        def _rare():
            cp = pltpu.make_async_copy(ce_hbm_ref, ce_scr, sem)
            cp.start()
            cp.wait()
            cr = min(512, rows_total)
            nch = rows_total // cr
            k_f = n_neg

            def count_ge(cand):
                def it(c, acc):
                    blk = ce_scr[pl.ds(c * cr, cr), :]
                    bits = jax.lax.bitcast_convert_type(blk, jnp.int32)
                    return acc + jnp.sum((bits >= cand).astype(jnp.float32))
                return jax.lax.fori_loop(0, nch, it, jnp.float32(0.0))

            def bit_it(i, t):
                cand = t | (jnp.int32(1) << (30 - i))
                return jnp.where(count_ge(cand) >= k_f, cand, t)

            t = jax.lax.fori_loop(0, 31, bit_it, jnp.int32(0))

            def fin_it(c, carry):
                s, n = carry
                blk = ce_scr[pl.ds(c * cr, cr), :]
                bits = jax.lax.bitcast_convert_type(blk, jnp.int32)
                gt = bits > t
                return (s + jnp.sum(jnp.where(gt, blk, 0.0)),
                        n + jnp.sum(gt.astype(jnp.float32)))

            s_gt, n_gt = jax.lax.fori_loop(
                0, nch, fin_it, (jnp.float32(0.0), jnp.float32(0.0)))
            t_val = jax.lax.bitcast_convert_type(t, jnp.float32)
            topk_ref[0] = s_gt + (k_f - n_gt) * t_val

        topk_sum = topk_ref[0]

        loss_tr = (loss_pos + topk_sum) / (n_pos + n_neg)
        loss_tcl = jnp.where(
            n_pos > 0.0, tcl_sum / jnp.maximum(n_pos, 1.0), 0.0)
        denom = jnp.maximum(n_tcl_sel, 1.0)
        cond = n_tcl_train > 0.0
        loss_radii = jnp.where(cond, radii_sum / denom, 0.0)
        loss_sin = jnp.where(cond, sin_sum / denom, 0.0)
        loss_cos = jnp.where(cond, cos_sum / denom, 0.0)

        gce = _ce2(gp_ref[0], gp_ref[1], gl_ref[...])
        gcn_loss = jnp.sum(gce) / jnp.float32(g_rows * LANES)

        outv = (jnp.where(lane1 == 0, loss_tr, 0.0)
                + jnp.where(lane1 == 1, loss_tcl, 0.0)
                + jnp.where(lane1 == 2, loss_sin, 0.0)
                + jnp.where(lane1 == 3, loss_cos, 0.0)
                + jnp.where(lane1 == 4, loss_radii, 0.0)
                + jnp.where(lane1 == 5, gcn_loss, 0.0))
        out_ref[...] = outv

    return body


def _combine(scp, tcp, gp, gl, ce_stash):
    rows_total = ce_stash.shape[0]
    g_rows = gp.shape[1]
    body = _make_combine_body(rows_total, g_rows)
    out = pl.pallas_call(
        body,
        in_specs=[
            pl.BlockSpec(memory_space=pltpu.MemorySpace.VMEM),
            pl.BlockSpec(memory_space=pltpu.MemorySpace.VMEM),
            pl.BlockSpec(memory_space=pltpu.MemorySpace.VMEM),
            pl.BlockSpec(memory_space=pltpu.MemorySpace.VMEM),
            pl.BlockSpec(memory_space=pltpu.MemorySpace.HBM),
        ],
        out_specs=pl.BlockSpec(memory_space=pltpu.MemorySpace.VMEM),
        out_shape=jax.ShapeDtypeStruct((1, LANES), jnp.float32),
        scratch_shapes=[
            pltpu.VMEM((rows_total, LANES), jnp.float32),
            pltpu.SMEM((1,), jnp.float32),
            pltpu.SemaphoreType.DMA,
        ],
    )(scp, tcp, gp, gl, ce_stash)
    return out


def kernel(inputs, gcn_pred, gcn_labels, train_mask, tr_mask, tcl_mask,
           radii_map, sin_map, cos_map):
    b, c, h, w = inputs.shape
    pix = h * w
    rows_b = pix // LANES
    g = gcn_pred.shape[0]
    g_rows = g // LANES

    x = inputs.reshape(b, 8, pix)
    trm_f = tr_mask.astype(jnp.int32).reshape(b, pix)
    tnm_f = train_mask.astype(jnp.int32).reshape(b, pix)

    sc_part, ce_stash = _sc_ohem(x, trm_f, tnm_f)

    x4 = inputs.reshape(b, 8, rows_b, LANES)
    trm = trm_f.reshape(b, rows_b, LANES)
    tnm = tnm_f.reshape(b, rows_b, LANES)
    tcl = tcl_mask.astype(jnp.int32).reshape(b, rows_b, LANES)
    rad = jnp.transpose(radii_map.reshape(b, pix, 2), (0, 2, 1))
    rad = rad.reshape(b, 2, rows_b, LANES)
    snm = sin_map.reshape(b, rows_b, LANES)
    csm = cos_map.reshape(b, rows_b, LANES)
    tc_part = _tc_dense(x4, trm, tnm, tcl, rad, snm, csm)

    gp = jnp.transpose(gcn_pred, (1, 0)).reshape(2, g_rows, LANES)
    gl = gcn_labels.astype(jnp.int32).reshape(g_rows, LANES)
    scp = sc_part.reshape(NW * 4 * 16 // LANES, LANES)
    ce2d = ce_stash.reshape(b * rows_b, LANES)

    out = _combine(scp, tc_part, gp, gl, ce2d)
    return (out[0, 0], out[0, 1], out[0, 2], out[0, 3], out[0, 4], out[0, 5])
